# baseline (speedup 1.0000x reference)
import functools

import jax
import jax.numpy as jnp
from jax import lax
from jax.experimental import pallas as pl
from jax.experimental.pallas import tpu as pltpu
from jax.experimental.pallas import tpu_sc as plsc

N = 10000
D = 128
G = 64

NC = 2
NS = 16
NW = NC * NS
CH = 256
ACC = 10240
RPT = ACC // NS

_mesh = plsc.VectorSubcoreMesh(core_axis_name="c", subcore_axis_name="s")


def _deg_body(nch, dst_hbm, ones_hbm, zeros_hbm, out_hbm, idx_v, ones_v,
              buf_v, acc, sem):
    c = lax.axis_index("c")
    s = lax.axis_index("s")
    wid = c * NS + s
    pltpu.sync_copy(zeros_hbm, buf_v)
    pltpu.sync_copy(buf_v, acc.at[pl.ds(s * RPT, RPT)])
    pltpu.sync_copy(ones_hbm, ones_v)
    plsc.subcore_barrier()
    base = wid * (nch * CH)

    def body(j, carry):
        pltpu.sync_copy(dst_hbm.at[pl.ds(base + j * CH, CH)], idx_v)
        pltpu.sync_copy(ones_v, acc.at[idx_v], add=True)
        return carry

    lax.fori_loop(0, nch, body, 0)
    plsc.subcore_barrier()
    pltpu.sync_copy(acc.at[pl.ds(s * RPT, RPT)], buf_v)
    pltpu.sync_copy(buf_v, out_hbm.at[c, pl.ds(s * RPT, RPT)])


def _deg_call(dst_p, nch):
    kern = functools.partial(
        pl.kernel,
        out_type=jax.ShapeDtypeStruct((NC, ACC), jnp.float32),
        mesh=_mesh,
        scratch_types=[
            pltpu.VMEM((CH,), jnp.int32),
            pltpu.VMEM((CH,), jnp.float32),
            pltpu.VMEM((RPT,), jnp.float32),
            pltpu.VMEM_SHARED((ACC,), jnp.float32),
            pltpu.SemaphoreType.DMA,
        ],
    )(functools.partial(_deg_body, nch))
    return kern(dst_p, jnp.ones((CH,), jnp.float32),
                jnp.zeros((RPT,), jnp.float32))


NBUF = 2


def _scat_body(nch, u_hbm, src_hbm, dst_hbm, zero_hbm, out_hbm,
               s0b, s1b, d0, d1, r0, acc,
               g0, g1, ss0, ss1, dd0, dd1, pp0, pp1):
    c = lax.axis_index("c")
    s = lax.axis_index("s")
    wid = c * NS + s
    sbufs = [s0b, s1b]
    dbufs = [d0, d1]
    rbufs = [r0]
    gsems = [g0, g1]
    ssems = [ss0, ss1]
    dsems = [dd0, dd1]
    psems = [pp0, pp1]
    z = r0.at[pl.ds(0, 128), :]
    pltpu.sync_copy(zero_hbm, z)
    for k in range(RPT // 128):
        pltpu.sync_copy(z, acc.at[pl.ds(s * RPT + k * 128, 128), :])
    plsc.subcore_barrier()
    ebase = wid * nch * CH

    def sload(j, b):
        return pltpu.make_async_copy(
            src_hbm.at[pl.ds(ebase + j * CH, CH)], sbufs[b], psems[b])

    def dload(j, b):
        return pltpu.make_async_copy(
            dst_hbm.at[pl.ds(ebase + j * CH, CH)], dbufs[b], dsems[b])

    def gath(b):
        return pltpu.make_async_copy(u_hbm.at[sbufs[b]], rbufs[b], gsems[b])

    def scat(b):
        return pltpu.make_async_copy(rbufs[b], acc.at[dbufs[b]], ssems[b])

    def chunk(j, carry):
        sload(j, 0).start()
        dload(j, 0).start()
        sload(j, 0).wait()
        dload(j, 0).wait()
        pltpu.async_copy(u_hbm.at[sbufs[0]], rbufs[0], gsems[0]).wait()
        pltpu.async_copy(rbufs[0], acc.at[dbufs[0]], ssems[0], add=True)
        scat(0).wait()
        return carry

    lax.fori_loop(0, nch, chunk, 0)
    plsc.subcore_barrier()
    for k in range(RPT // 128):
        pltpu.sync_copy(acc.at[pl.ds(s * RPT + k * 128, 128), :], z)
        pltpu.sync_copy(z, out_hbm.at[c, pl.ds(s * RPT + k * 128, 128), :])


def _scat_call(u, src_p, dst_p, nch):
    kern = functools.partial(
        pl.kernel,
        out_type=jax.ShapeDtypeStruct((NC, ACC, D), jnp.float32),
        mesh=_mesh,
        scratch_types=[
            pltpu.VMEM((CH,), jnp.int32),
            pltpu.VMEM((CH,), jnp.int32),
            pltpu.VMEM((CH,), jnp.int32),
            pltpu.VMEM((CH,), jnp.int32),
            pltpu.VMEM((CH, D), jnp.float32),
            pltpu.VMEM_SHARED((ACC, D), jnp.float32),
        ] + [pltpu.SemaphoreType.DMA] * 8,
    )(functools.partial(_scat_body, nch))
    return kern(u, src_p, dst_p, jnp.zeros((128, D), jnp.float32))


def _mm1_body(x_ref, w_ref, dinv_ref, o_ref):
    o_ref[...] = jnp.dot(x_ref[...], w_ref[...],
                         preferred_element_type=jnp.float32) * dinv_ref[...]


def _mmc_body(s0_ref, s1_ref, u_ref, dinv_ref, b_ref, w_ref, o_ref):
    h = (s0_ref[...] + s1_ref[...] + u_ref[...]) * dinv_ref[...] + b_ref[...]
    h = jax.nn.relu(h)
    o_ref[...] = jnp.dot(h, w_ref[...],
                         preferred_element_type=jnp.float32) * dinv_ref[...]


def _head_body(starts_ref, s0_ref, s1_ref, u_ref, dinv_ref, b_ref,
               lw1_ref, lb1_ref, lw2_ref, lb2_ref, lw3_ref, lb3_ref,
               o_ref, h3_ref, x1_ref, x2_ref):
    nb = 80

    def fill(i, carry):
        sl = pl.ds(i * nb, nb)
        h3_ref[sl, :] = ((s0_ref[sl, :] + s1_ref[sl, :] + u_ref[sl, :])
                         * dinv_ref[sl, :] + b_ref[...])
        return carry

    lax.fori_loop(0, N // nb, fill, 0)

    def seg(g, carry):
        start = starts_ref[g]
        end = starts_ref[g + 1]
        lo = lax.div(start, 8)
        hi = lax.div(end + 7, 8)

        def blk(i, sm):
            sacc, macc = sm
            rows = h3_ref[pl.ds(i * 8, 8), :]
            rid = i * 8 + lax.broadcasted_iota(jnp.int32, (8, 1), 0)
            mask = (rid >= start) & (rid < end)
            sacc = sacc + jnp.sum(jnp.where(mask, rows, 0.0), axis=0,
                                  keepdims=True)
            macc = jnp.maximum(macc, jnp.max(
                jnp.where(mask, rows, -jnp.inf), axis=0, keepdims=True))
            return (sacc, macc)

        sacc, macc = lax.fori_loop(
            lo, hi, blk,
            (jnp.zeros((1, D), jnp.float32),
             jnp.full((1, D), -jnp.inf, jnp.float32)))
        x1_ref[pl.ds(g, 1), :] = sacc
        x2_ref[pl.ds(g, 1), :] = jnp.where(macc == -jnp.inf, 0.0, macc)
        return carry

    lax.fori_loop(0, G, seg, 0)

    z = jnp.concatenate([x1_ref[...], x2_ref[...]], axis=1)
    z = jax.nn.relu(jnp.dot(z, lw1_ref[...],
                            preferred_element_type=jnp.float32) + lb1_ref[...])
    z = jax.nn.relu(jnp.dot(z, lw2_ref[...],
                            preferred_element_type=jnp.float32) + lb2_ref[...])
    o_ref[...] = jnp.dot(z, lw3_ref[...],
                         preferred_element_type=jnp.float32) + lb3_ref[...]


def _head_call(starts, s0, s1, u, dinv, b3, lw1, lb1, lw2, lb2, lw3, lb3):
    n_in = 12
    specs = [pl.BlockSpec(memory_space=pltpu.SMEM)]
    specs += [pl.BlockSpec(memory_space=pltpu.VMEM)] * (n_in - 1)
    return pl.pallas_call(
        _head_body,
        out_shape=jax.ShapeDtypeStruct((G, 1), jnp.float32),
        in_specs=specs,
        out_specs=pl.BlockSpec(memory_space=pltpu.VMEM),
        scratch_shapes=[
            pltpu.VMEM((N, D), jnp.float32),
            pltpu.VMEM((G, D), jnp.float32),
            pltpu.VMEM((G, D), jnp.float32),
        ],
    )(starts, s0, s1, u, dinv, b3, lw1, lb1, lw2, lb2, lw3, lb3)


def kernel(x, edge_index, edge_attr, batch, W1, b1, W2, b2, W3, b3,
           lw1, lb1, lw2, lb2, lw3, lb3):
    e = edge_index.shape[1]
    quant = NW * CH
    epad = -(-e // quant) * quant
    nch = epad // (NW * CH)
    pad = epad - e
    src_p = jnp.concatenate(
        [edge_index[0], jnp.zeros((pad,), edge_index.dtype)]).astype(jnp.int32)
    dst_p = jnp.concatenate(
        [edge_index[1], jnp.full((pad,), N, edge_index.dtype)]).astype(jnp.int32)

    deg2 = _deg_call(dst_p, nch)
    deg = deg2[0, :N] + deg2[1, :N] + 1.0
    dinv = lax.rsqrt(deg).reshape(N, 1)

    starts = jnp.searchsorted(
        batch, jnp.arange(G + 1, dtype=batch.dtype)).astype(jnp.int32)

    mm1 = pl.pallas_call(
        _mm1_body, out_shape=jax.ShapeDtypeStruct((N, D), jnp.float32))
    mmc = pl.pallas_call(
        _mmc_body, out_shape=jax.ShapeDtypeStruct((N, D), jnp.float32))

    u1 = mm1(x, W1, dinv)
    s1 = _scat_call(u1, src_p, dst_p, nch)
    u2 = mmc(s1[0, :N], s1[1, :N], u1, dinv, b1.reshape(1, D), W2)
    s2 = _scat_call(u2, src_p, dst_p, nch)
    u3 = mmc(s2[0, :N], s2[1, :N], u2, dinv, b2.reshape(1, D), W3)
    s3 = _scat_call(u3, src_p, dst_p, nch)

    return _head_call(starts, s3[0, :N], s3[1, :N], u3, dinv,
                      b3.reshape(1, D), lw1, lb1.reshape(1, 64),
                      lw2, lb2.reshape(1, 64), lw3, lb3.reshape(1, 1))


# speedup vs baseline: 1.3036x; 1.3036x over previous
import functools

import jax
import jax.numpy as jnp
from jax import lax
from jax.experimental import pallas as pl
from jax.experimental.pallas import tpu as pltpu
from jax.experimental.pallas import tpu_sc as plsc

N = 10000
D = 128
G = 64

NC = 2
NS = 16
NW = NC * NS
CH = 128
ACC = 10240
RPT = ACC // NS

_mesh = plsc.VectorSubcoreMesh(core_axis_name="c", subcore_axis_name="s")


def _deg_body(nch, dst_hbm, ones_hbm, zeros_hbm, out_hbm, idx_v, ones_v,
              buf_v, acc, sem):
    c = lax.axis_index("c")
    s = lax.axis_index("s")
    wid = c * NS + s
    pltpu.sync_copy(zeros_hbm, buf_v)
    pltpu.sync_copy(buf_v, acc.at[pl.ds(s * RPT, RPT)])
    pltpu.sync_copy(ones_hbm, ones_v)
    plsc.subcore_barrier()
    base = wid * (nch * CH)

    def body(j, carry):
        pltpu.sync_copy(dst_hbm.at[pl.ds(base + j * CH, CH)], idx_v)
        pltpu.sync_copy(ones_v, acc.at[idx_v], add=True)
        return carry

    lax.fori_loop(0, nch, body, 0)
    plsc.subcore_barrier()
    pltpu.sync_copy(acc.at[pl.ds(s * RPT, RPT)], buf_v)
    pltpu.sync_copy(buf_v, out_hbm.at[c, pl.ds(s * RPT, RPT)])


def _deg_call(dst_p, nch):
    kern = functools.partial(
        pl.kernel,
        out_type=jax.ShapeDtypeStruct((NC, ACC), jnp.float32),
        mesh=_mesh,
        scratch_types=[
            pltpu.VMEM((CH,), jnp.int32),
            pltpu.VMEM((CH,), jnp.float32),
            pltpu.VMEM((RPT,), jnp.float32),
            pltpu.VMEM_SHARED((ACC,), jnp.float32),
            pltpu.SemaphoreType.DMA,
        ],
    )(functools.partial(_deg_body, nch))
    return kern(dst_p, jnp.ones((CH,), jnp.float32),
                jnp.zeros((RPT,), jnp.float32))


NBUF = 2


def _scat_body(nch, u_hbm, src_hbm, dst_hbm, zero_hbm, out_hbm,
               s0b, s1b, d0, d1, r0, r1, acc,
               g0, g1, ss0, ss1, dd0, dd1, pp0, pp1):
    c = lax.axis_index("c")
    s = lax.axis_index("s")
    wid = c * NS + s
    sbufs = [s0b, s1b]
    dbufs = [d0, d1]
    rbufs = [r0, r1]
    gsems = [g0, g1]
    ssems = [ss0, ss1]
    dsems = [dd0, dd1]
    psems = [pp0, pp1]
    pltpu.sync_copy(zero_hbm, r0)
    for k in range(RPT // CH):
        pltpu.sync_copy(r0, acc.at[pl.ds(s * RPT + k * CH, CH), :])
    plsc.subcore_barrier()
    ebase = wid * nch * CH

    def sload(j, b):
        return pltpu.make_async_copy(
            src_hbm.at[pl.ds(ebase + j * CH, CH)], sbufs[b], psems[b])

    def dload(j, b):
        return pltpu.make_async_copy(
            dst_hbm.at[pl.ds(ebase + j * CH, CH)], dbufs[b], dsems[b])

    def gath(b):
        return pltpu.make_async_copy(u_hbm.at[sbufs[b]], rbufs[b], gsems[b])

    def scat(b):
        return pltpu.make_async_copy(rbufs[b], acc.at[dbufs[b]], ssems[b])

    def chunk(j, carry):
        sload(j, 0).start()
        dload(j, 0).start()
        sload(j, 0).wait()
        dload(j, 0).wait()
        pltpu.async_copy(u_hbm.at[sbufs[0]], rbufs[0], gsems[0]).wait()
        pltpu.async_copy(rbufs[0], acc.at[dbufs[0]], ssems[0], add=True)
        scat(0).wait()
        return carry

    lax.fori_loop(0, nch, chunk, 0)
    plsc.subcore_barrier()
    for k in range(RPT // CH):
        pltpu.sync_copy(acc.at[pl.ds(s * RPT + k * CH, CH), :], r0)
        pltpu.sync_copy(r0, out_hbm.at[c, pl.ds(s * RPT + k * CH, CH), :])


def _scat_call(u, src_p, dst_p, nch):
    kern = functools.partial(
        pl.kernel,
        out_type=jax.ShapeDtypeStruct((NC, ACC, D), jnp.float32),
        mesh=_mesh,
        scratch_types=[
            pltpu.VMEM((CH,), jnp.int32),
            pltpu.VMEM((CH,), jnp.int32),
            pltpu.VMEM((CH,), jnp.int32),
            pltpu.VMEM((CH,), jnp.int32),
            pltpu.VMEM((CH, D), jnp.float32),
            pltpu.VMEM((CH, D), jnp.float32),
            pltpu.VMEM_SHARED((ACC, D), jnp.float32),
        ] + [pltpu.SemaphoreType.DMA] * 8,
    )(functools.partial(_scat_body, nch))
    return kern(u, src_p, dst_p, jnp.zeros((CH, D), jnp.float32))


def _mm1_body(x_ref, w_ref, dinv_ref, o_ref):
    o_ref[...] = jnp.dot(x_ref[...], w_ref[...],
                         preferred_element_type=jnp.float32) * dinv_ref[...]


def _mmc_body(s0_ref, s1_ref, u_ref, dinv_ref, b_ref, w_ref, o_ref):
    h = (s0_ref[...] + s1_ref[...] + u_ref[...]) * dinv_ref[...] + b_ref[...]
    h = jax.nn.relu(h)
    o_ref[...] = jnp.dot(h, w_ref[...],
                         preferred_element_type=jnp.float32) * dinv_ref[...]


def _head_body(starts_ref, s0_ref, s1_ref, u_ref, dinv_ref, b_ref,
               lw1_ref, lb1_ref, lw2_ref, lb2_ref, lw3_ref, lb3_ref,
               o_ref, h3_ref, x1_ref, x2_ref):
    nb = 80

    def fill(i, carry):
        sl = pl.ds(i * nb, nb)
        h3_ref[sl, :] = ((s0_ref[sl, :] + s1_ref[sl, :] + u_ref[sl, :])
                         * dinv_ref[sl, :] + b_ref[...])
        return carry

    lax.fori_loop(0, N // nb, fill, 0)

    def seg(g, carry):
        start = starts_ref[g]
        end = starts_ref[g + 1]
        lo = lax.div(start, 8)
        hi = lax.div(end + 7, 8)

        def blk(i, sm):
            sacc, macc = sm
            rows = h3_ref[pl.ds(i * 8, 8), :]
            rid = i * 8 + lax.broadcasted_iota(jnp.int32, (8, 1), 0)
            mask = (rid >= start) & (rid < end)
            sacc = sacc + jnp.sum(jnp.where(mask, rows, 0.0), axis=0,
                                  keepdims=True)
            macc = jnp.maximum(macc, jnp.max(
                jnp.where(mask, rows, -jnp.inf), axis=0, keepdims=True))
            return (sacc, macc)

        sacc, macc = lax.fori_loop(
            lo, hi, blk,
            (jnp.zeros((1, D), jnp.float32),
             jnp.full((1, D), -jnp.inf, jnp.float32)))
        x1_ref[pl.ds(g, 1), :] = sacc
        x2_ref[pl.ds(g, 1), :] = jnp.where(macc == -jnp.inf, 0.0, macc)
        return carry

    lax.fori_loop(0, G, seg, 0)

    z = jnp.concatenate([x1_ref[...], x2_ref[...]], axis=1)
    z = jax.nn.relu(jnp.dot(z, lw1_ref[...],
                            preferred_element_type=jnp.float32) + lb1_ref[...])
    z = jax.nn.relu(jnp.dot(z, lw2_ref[...],
                            preferred_element_type=jnp.float32) + lb2_ref[...])
    o_ref[...] = jnp.dot(z, lw3_ref[...],
                         preferred_element_type=jnp.float32) + lb3_ref[...]


def _head_call(starts, s0, s1, u, dinv, b3, lw1, lb1, lw2, lb2, lw3, lb3):
    n_in = 12
    specs = [pl.BlockSpec(memory_space=pltpu.SMEM)]
    specs += [pl.BlockSpec(memory_space=pltpu.VMEM)] * (n_in - 1)
    return pl.pallas_call(
        _head_body,
        out_shape=jax.ShapeDtypeStruct((G, 1), jnp.float32),
        in_specs=specs,
        out_specs=pl.BlockSpec(memory_space=pltpu.VMEM),
        scratch_shapes=[
            pltpu.VMEM((N, D), jnp.float32),
            pltpu.VMEM((G, D), jnp.float32),
            pltpu.VMEM((G, D), jnp.float32),
        ],
    )(starts, s0, s1, u, dinv, b3, lw1, lb1, lw2, lb2, lw3, lb3)


def kernel(x, edge_index, edge_attr, batch, W1, b1, W2, b2, W3, b3,
           lw1, lb1, lw2, lb2, lw3, lb3):
    e = edge_index.shape[1]
    quant = NW * CH
    epad = -(-e // quant) * quant
    nch = epad // (NW * CH)
    pad = epad - e
    src_p = jnp.concatenate(
        [edge_index[0], jnp.zeros((pad,), edge_index.dtype)]).astype(jnp.int32)
    dst_p = jnp.concatenate(
        [edge_index[1], jnp.full((pad,), N, edge_index.dtype)]).astype(jnp.int32)

    deg2 = _deg_call(dst_p, nch)
    deg = deg2[0, :N] + deg2[1, :N] + 1.0
    dinv = lax.rsqrt(deg).reshape(N, 1)

    starts = jnp.searchsorted(
        batch, jnp.arange(G + 1, dtype=batch.dtype)).astype(jnp.int32)

    mm1 = pl.pallas_call(
        _mm1_body, out_shape=jax.ShapeDtypeStruct((N, D), jnp.float32))
    mmc = pl.pallas_call(
        _mmc_body, out_shape=jax.ShapeDtypeStruct((N, D), jnp.float32))

    u1 = mm1(x, W1, dinv)
    s1 = _scat_call(u1, src_p, dst_p, nch)
    u2 = mmc(s1[0, :N], s1[1, :N], u1, dinv, b1.reshape(1, D), W2)
    s2 = _scat_call(u2, src_p, dst_p, nch)
    u3 = mmc(s2[0, :N], s2[1, :N], u2, dinv, b2.reshape(1, D), W3)
    s3 = _scat_call(u3, src_p, dst_p, nch)

    return _head_call(starts, s3[0, :N], s3[1, :N], u3, dinv,
                      b3.reshape(1, D), lw1, lb1.reshape(1, 64),
                      lw2, lb2.reshape(1, 64), lw3, lb3.reshape(1, 1))
